# Initial kernel scaffold; baseline (speedup 1.0000x reference)
import jax
import jax.numpy as jnp
from jax.experimental import pallas as pl


def kernel(features, src, dst, rel, alignment, train_year, basis0, comb0, Wself0, Wtime0, bias0, basis1, comb1, Wself1, Wtime1, bias1):
    raise NotImplementedError("write your pallas kernel here")


# speedup vs baseline: 13.7644x; 13.7644x over previous
import functools

import jax
import jax.numpy as jnp
from jax import lax
from jax.experimental import pallas as pl
from jax.experimental.pallas import tpu as pltpu
import jax.experimental.pallas.tpu_sc as plsc

_NC = 2
_NS = 16
_CH = 128


def _sc_gather_scatter(table_rows, n_rows, d, nch):
    np_rows = ((n_rows + 1 + 127) // 128) * 128
    zper = np_rows // _NS
    mesh = plsc.VectorSubcoreMesh(core_axis_name="c", subcore_axis_name="s")

    @functools.partial(
        pl.kernel,
        mesh=mesh,
        out_type=jax.ShapeDtypeStruct((_NC, np_rows, d), jnp.float32),
        scratch_types=[
            pltpu.VMEM((nch, _CH), jnp.int32),
            pltpu.VMEM((nch, _CH), jnp.int32),
            pltpu.VMEM((_CH, d), jnp.float32),
            pltpu.VMEM_SHARED((np_rows, d), jnp.float32),
            pltpu.SemaphoreType.DMA,
        ],
    )
    def k(table, gidx, didx, zrows, out, gv, dv, rows, agg, sem):
        c = lax.axis_index("c")
        s = lax.axis_index("s")
        pltpu.sync_copy(zrows, agg.at[pl.ds(s * zper, zper)])
        pltpu.sync_copy(gidx.at[c, s], gv)
        pltpu.sync_copy(didx.at[c, s], dv)
        plsc.subcore_barrier()

        def chunk(j, carry):
            pltpu.async_copy(table.at[gv.at[j]], rows, sem).wait()
            pltpu.sync_copy(rows, agg.at[dv.at[j]], add=True)
            return carry

        lax.fori_loop(0, nch, chunk, 0)
        plsc.subcore_barrier()
        pltpu.sync_copy(agg.at[pl.ds(s * zper, zper)],
                        out.at[c, pl.ds(s * zper, zper)])

    return k


def _pack_idx(gi, di, nch, trash):
    k = gi.shape[0]
    tot = _NC * _NS * nch * _CH
    gp = jnp.zeros((tot,), jnp.int32).at[:k].set(gi.astype(jnp.int32))
    dp = jnp.full((tot,), trash, jnp.int32).at[:k].set(di.astype(jnp.int32))
    return gp.reshape(_NC, _NS, nch, _CH), dp.reshape(_NC, _NS, nch, _CH)


def _tc_hx(x, w2, wself, bias, bn=2000):
    n, d = x.shape
    rd = w2.shape[1]

    def body(x_ref, w2_ref, ws_ref, b_ref, h_ref, s_ref):
        xb = x_ref[...]
        h_ref[...] = jnp.dot(xb, w2_ref[...], preferred_element_type=jnp.float32)
        s_ref[...] = (jnp.dot(xb, ws_ref[...], preferred_element_type=jnp.float32)
                      + b_ref[...])

    return pl.pallas_call(
        body,
        grid=(n // bn,),
        in_specs=[
            pl.BlockSpec((bn, d), lambda i: (i, 0)),
            pl.BlockSpec((d, rd), lambda i: (0, 0)),
            pl.BlockSpec((d, d), lambda i: (0, 0)),
            pl.BlockSpec((1, d), lambda i: (0, 0)),
        ],
        out_specs=[
            pl.BlockSpec((bn, rd), lambda i: (i, 0)),
            pl.BlockSpec((bn, d), lambda i: (i, 0)),
        ],
        out_shape=[
            jax.ShapeDtypeStruct((n, rd), jnp.float32),
            jax.ShapeDtypeStruct((n, d), jnp.float32),
        ],
    )(x, w2, wself, bias.reshape(1, d))


def _tc_combine(parts, xself, gparts, wt, relu, bn=2000):
    n, d = xself.shape

    if gparts is None:
        def body(p_ref, s_ref, o_ref):
            o = p_ref[0] + p_ref[1] + s_ref[...]
            o_ref[...] = jnp.maximum(o, 0.0) if relu else o

        return pl.pallas_call(
            body,
            grid=(n // bn,),
            in_specs=[
                pl.BlockSpec((2, bn, d), lambda i: (0, i, 0)),
                pl.BlockSpec((bn, d), lambda i: (i, 0)),
            ],
            out_specs=pl.BlockSpec((bn, d), lambda i: (i, 0)),
            out_shape=jax.ShapeDtypeStruct((n, d), jnp.float32),
        )(parts, xself)

    def body(p_ref, s_ref, g_ref, wt_ref, o_ref):
        g = g_ref[0] + g_ref[1]
        o = (p_ref[0] + p_ref[1] + s_ref[...]
             + jnp.dot(g, wt_ref[...], preferred_element_type=jnp.float32))
        o_ref[...] = jnp.maximum(o, 0.0) if relu else o

    return pl.pallas_call(
        body,
        grid=(n // bn,),
        in_specs=[
            pl.BlockSpec((2, bn, d), lambda i: (0, i, 0)),
            pl.BlockSpec((bn, d), lambda i: (i, 0)),
            pl.BlockSpec((2, bn, d), lambda i: (0, i, 0)),
            pl.BlockSpec((d, d), lambda i: (0, 0)),
        ],
        out_specs=pl.BlockSpec((bn, d), lambda i: (i, 0)),
        out_shape=jax.ShapeDtypeStruct((n, d), jnp.float32),
    )(parts, xself, gparts, wt)


def kernel(features, src, dst, rel, alignment, train_year,
           basis0, comb0, Wself0, Wtime0, bias0,
           basis1, comb1, Wself1, Wtime1, bias1):
    t_years, n, d = features.shape
    r = comb0.shape[0]
    e = src.shape[1]

    nch_e = -(-e // (_NC * _NS * _CH))
    nch_a = -(-n // (_NC * _NS * _CH))
    np_rows = ((n + 1 + 127) // 128) * 128
    zrows = jnp.zeros((np_rows // _NS, d), jnp.float32)

    sc_edge = _sc_gather_scatter(n * r, n, d, nch_e)
    sc_align = _sc_gather_scatter(n, n, d, nch_a)

    layers = []
    for basis, comb, wself, wtime, bias in (
            (basis0, comb0, Wself0, Wtime0, bias0),
            (basis1, comb1, Wself1, Wtime1, bias1)):
        w2 = jnp.einsum('rb,bdo->dro', comb, basis).reshape(d, r * d)
        layers.append((w2, wself, wtime, bias))

    embs = []
    for t in range(t_years):
        x = features[t]
        gidx, didx = _pack_idx(src[t] * r + rel[t], dst[t], nch_e, n)
        gparts = None
        if t > 0:
            ga, da = _pack_idx(alignment[:, t - 1], alignment[:, t], nch_a, n)
            gparts = sc_align(embs[-1], ga, da, zrows)
        for i, (w2, wself, wtime, bias) in enumerate(layers):
            h, xself = _tc_hx(x, w2, wself, bias)
            parts = sc_edge(h.reshape(n * r, d), gidx, didx, zrows)
            x = _tc_combine(parts, xself, gparts, wtime,
                            relu=(i != len(layers) - 1))
        embs.append(x)

    stacked = jnp.stack(embs)
    mask = (jnp.arange(t_years) < train_year)[:, None, None]
    return jnp.where(mask, stacked, jnp.zeros_like(stacked))
